# baseline (speedup 1.0000x reference)
import functools

import jax
import jax.numpy as jnp
from jax import lax
from jax.experimental import pallas as pl
from jax.experimental.pallas import tpu as pltpu
from jax.experimental.pallas import tpu_sc as plsc

B = 16384
D = 64
L = 16

_info = plsc.get_sparse_core_info()
NC = _info.num_cores
NS = _info.num_subcores
NW = NC * NS
BW = B // NW

_mesh = plsc.VectorSubcoreMesh(core_axis_name="c", subcore_axis_name="s")


@functools.partial(
    pl.kernel,
    out_type=jax.ShapeDtypeStruct((B,), jnp.float32),
    mesh=_mesh,
    compiler_params=pltpu.CompilerParams(needs_layout_passes=False,
                                         use_tc_tiling_on_sc=False),
    scratch_types=[
        pltpu.VMEM((BW,), jnp.int32),
        pltpu.VMEM((BW,), jnp.int32),
        pltpu.VMEM((BW, D), jnp.float32),
        pltpu.VMEM((BW, D), jnp.float32),
        pltpu.VMEM((BW, L), jnp.float32),
        pltpu.VMEM((BW, L), jnp.float32),
        pltpu.VMEM((BW,), jnp.int32),
        pltpu.VMEM((BW,), jnp.int32),
        pltpu.VMEM((BW,), jnp.float32),
        pltpu.VMEM((L,), jnp.float32),
        pltpu.SemaphoreType.DMA,
        pltpu.SemaphoreType.DMA,
        pltpu.SemaphoreType.DMA,
        pltpu.SemaphoreType.DMA,
    ],
)
def _mf_kernel(uid_hbm, iid_hbm, uemb_hbm, iemb_hbm, ub_hbm, ib_hbm, gb_hbm,
               out_hbm, uid_v, iid_v, urows, irows, ub_v, ib_v, ubx_v, ibx_v,
               dots, gb_v, sem_u, sem_i, sem_ub, sem_ib):
    wid = lax.axis_index("s") * NC + lax.axis_index("c")
    base = wid * BW
    lane = lax.iota(jnp.int32, L)

    pltpu.sync_copy(uid_hbm.at[pl.ds(base, BW)], uid_v)
    pltpu.sync_copy(iid_hbm.at[pl.ds(base, BW)], iid_v)

    cp_u = pltpu.async_copy(uemb_hbm.at[uid_v], urows, sem_u)
    cp_i = pltpu.async_copy(iemb_hbm.at[iid_v], irows, sem_i)

    def bidx_body(j, _):
        o = j * L
        ubx_v[pl.ds(o, L)] = lax.shift_right_logical(uid_v[pl.ds(o, L)], 4)
        ibx_v[pl.ds(o, L)] = lax.shift_right_logical(iid_v[pl.ds(o, L)], 4)
        return 0

    lax.fori_loop(0, BW // L, bidx_body, 0, unroll=4)

    cp_ub = pltpu.async_copy(ub_hbm.at[ubx_v], ub_v, sem_ub)
    cp_ib = pltpu.async_copy(ib_hbm.at[ibx_v], ib_v, sem_ib)
    pltpu.sync_copy(gb_hbm, gb_v)

    cp_u.wait()
    cp_i.wait()

    def grp_body(g, _):
        res = jnp.zeros((L,), jnp.float32)
        for k in range(L):
            r = g * L + k
            acc = urows[r, pl.ds(0, L)] * irows[r, pl.ds(0, L)]
            for c in range(1, D // L):
                acc = acc + urows[r, pl.ds(c * L, L)] * irows[r, pl.ds(c * L, L)]
            res = jnp.where(lane == k, jnp.sum(acc), res)
        dots[pl.ds(g * L, L)] = res
        return 0

    lax.fori_loop(0, BW // L, grp_body, 0)

    cp_ub.wait()
    cp_ib.wait()
    gb = gb_v[...]

    def bias_body(j, _):
        o = j * L
        ub = plsc.load_gather(ub_v, [o + lane, uid_v[pl.ds(o, L)] & 15])
        ib = plsc.load_gather(ib_v, [o + lane, iid_v[pl.ds(o, L)] & 15])
        dots[pl.ds(o, L)] = dots[pl.ds(o, L)] + ub + ib + gb
        return 0

    lax.fori_loop(0, BW // L, bias_body, 0, unroll=4)

    pltpu.sync_copy(dots, out_hbm.at[pl.ds(base, BW)])


def kernel(user_ids, item_ids, user_emb_table, item_emb_table,
           user_bias_table, item_bias_table, global_bias):
    gb16 = jnp.broadcast_to(global_bias.astype(jnp.float32), (L,))
    ub16 = jnp.reshape(user_bias_table, (-1, L))
    ib16 = jnp.reshape(item_bias_table, (-1, L))
    return _mf_kernel(user_ids.astype(jnp.int32), item_ids.astype(jnp.int32),
                      user_emb_table, item_emb_table, ub16, ib16, gb16)


# speedup vs baseline: 1.0010x; 1.0010x over previous
import functools

import jax
import jax.numpy as jnp
from jax import lax
from jax.experimental import pallas as pl
from jax.experimental.pallas import tpu as pltpu
from jax.experimental.pallas import tpu_sc as plsc

B = 16384
D = 64
L = 16
NROW = 1000000
NCH = NROW // L

_info = plsc.get_sparse_core_info()
NC = _info.num_cores
NS = _info.num_subcores
NW = NC * NS
BW = B // NW
NG = BW // L

_mesh = plsc.VectorSubcoreMesh(core_axis_name="c", subcore_axis_name="s")


@functools.partial(
    pl.kernel,
    out_type=jax.ShapeDtypeStruct((B,), jnp.float32),
    mesh=_mesh,
    compiler_params=pltpu.CompilerParams(needs_layout_passes=False,
                                         use_tc_tiling_on_sc=False),
    scratch_types=[
        pltpu.VMEM((BW,), jnp.int32),
        pltpu.VMEM((BW,), jnp.int32),
        pltpu.VMEM((BW,), jnp.int32),
        pltpu.VMEM((BW,), jnp.int32),
        pltpu.VMEM((BW,), jnp.int32),
        pltpu.VMEM((BW,), jnp.int32),
        pltpu.VMEM((BW, D), jnp.float32),
        pltpu.VMEM((BW, D), jnp.float32),
        pltpu.VMEM((BW, L), jnp.float32),
        pltpu.VMEM((BW, L), jnp.float32),
        pltpu.VMEM((BW,), jnp.float32),
        pltpu.VMEM((L,), jnp.float32),
        pltpu.SemaphoreType.DMA,
        pltpu.SemaphoreType.DMA,
        pltpu.SemaphoreType.DMA,
        pltpu.SemaphoreType.DMA,
    ],
)
def _mf_kernel(uid_hbm, iid_hbm, uemb_hbm, iemb_hbm, ub_hbm, ib_hbm, gb_hbm,
               out_hbm, uid_v, iid_v, ux_v, ix_v, us_v, is_v, urows, irows,
               ub_v, ib_v, dots, gb_v, sem_u, sem_i, sem_ub, sem_ib):
    wid = lax.axis_index("s") * NC + lax.axis_index("c")
    base = wid * BW
    lane = lax.iota(jnp.int32, L)

    pltpu.sync_copy(uid_hbm.at[pl.ds(base, BW)], uid_v)
    pltpu.sync_copy(iid_hbm.at[pl.ds(base, BW)], iid_v)

    cp_u = pltpu.async_copy(uemb_hbm.at[uid_v], urows, sem_u)
    cp_i = pltpu.async_copy(iemb_hbm.at[iid_v], irows, sem_i)

    def prep_body(j, _):
        o = j * L
        u = uid_v[pl.ds(o, L)]
        i = iid_v[pl.ds(o, L)]
        ux_v[pl.ds(o, L)] = lax.shift_right_logical(u, 4)
        ix_v[pl.ds(o, L)] = lax.shift_right_logical(i, 4)
        us_v[pl.ds(o, L)] = u & 15
        is_v[pl.ds(o, L)] = i & 15
        return 0

    lax.fori_loop(0, NG, prep_body, 0, unroll=4)

    cp_ub = pltpu.async_copy(ub_hbm.at[ux_v], ub_v, sem_ub)
    cp_ib = pltpu.async_copy(ib_hbm.at[ix_v], ib_v, sem_ib)
    pltpu.sync_copy(gb_hbm, gb_v)

    cp_u.wait()
    cp_i.wait()

    def grp_body(g, _):
        res = jnp.zeros((L,), jnp.float32)
        for k in range(L):
            r = g * L + k
            acc = urows[r, pl.ds(0, L)] * irows[r, pl.ds(0, L)]
            for c in range(1, D // L):
                acc = acc + urows[r, pl.ds(c * L, L)] * irows[r, pl.ds(c * L, L)]
            res = jnp.where(lane == k, jnp.sum(acc), res)
        dots[pl.ds(g * L, L)] = res
        return 0

    lax.fori_loop(0, NG, grp_body, 0)

    cp_ub.wait()
    cp_ib.wait()
    gb = gb_v[...]

    def bias_body(j, _):
        o = j * L
        rows = o + lane
        ub = plsc.load_gather(ub_v, [rows, us_v[pl.ds(o, L)]])
        ib = plsc.load_gather(ib_v, [rows, is_v[pl.ds(o, L)]])
        dots[pl.ds(o, L)] = dots[pl.ds(o, L)] + ub + ib + gb
        return 0

    lax.fori_loop(0, NG, bias_body, 0, unroll=4)

    pltpu.sync_copy(dots, out_hbm.at[pl.ds(base, BW)])


def kernel(user_ids, item_ids, user_emb_table, item_emb_table,
           user_bias_table, item_bias_table, global_bias):
    ub16 = jnp.reshape(jnp.transpose(user_bias_table), (NCH, L))
    ib16 = jnp.reshape(jnp.transpose(item_bias_table), (NCH, L))
    gb16 = jnp.broadcast_to(global_bias.astype(jnp.float32), (L,))
    return _mf_kernel(user_ids.astype(jnp.int32), item_ids.astype(jnp.int32),
                      user_emb_table, item_emb_table, ub16, ib16, gb16)
